# Initial kernel scaffold; baseline (speedup 1.0000x reference)
import functools

import jax
import jax.numpy as jnp
from jax import lax
from jax.experimental import pallas as pl
from jax.experimental.pallas import tpu as pltpu
from jax.experimental.pallas import tpu_sc as plsc

NC = 2
NS = 16
LN = 16
K = 128


def _mm2_body(h_ref, w1_ref, w2_ref, o1_ref, o2_ref):
    x = h_ref[...]
    dn = (((1,), (1,)), ((), ()))
    o1_ref[...] = lax.dot_general(x, w1_ref[...], dn,
                                  preferred_element_type=jnp.float32)
    o2_ref[...] = lax.dot_general(x, w2_ref[...], dn,
                                  preferred_element_type=jnp.float32)


def _update_body(h_ref, hu_ref, agg_ref, wb_ref, b_ref, g_ref, be_ref, o_ref):
    dn = (((1,), (1,)), ((), ()))
    u = hu_ref[...] + lax.dot_general(agg_ref[...], wb_ref[...], dn,
                                      preferred_element_type=jnp.float32)
    u = jnp.maximum(u + b_ref[...], 0.0)
    x = h_ref[...] + u
    mu = jnp.mean(x, axis=-1, keepdims=True)
    var = jnp.mean(jnp.square(x - mu), axis=-1, keepdims=True)
    o_ref[...] = (x - mu) * lax.rsqrt(var + 1e-5) * g_ref[...] + be_ref[...]


def _make_sc_kernel(N, D, EPT, ACC_ROWS):
    CH = EPT // K
    NSEG = D // LN
    ZCH = ACC_ROWS // (NS * K)
    S_OUT = (N // (8 * NS)) * 8
    TAIL = N - NS * S_OUT
    mesh = plsc.VectorSubcoreMesh(core_axis_name="c", subcore_axis_name="s")

    @functools.partial(
        pl.kernel,
        out_type=jax.ShapeDtypeStruct((NC, N, D), jnp.float32),
        mesh=mesh,
        scratch_types=[
            pltpu.VMEM_SHARED((ACC_ROWS, D), jnp.float32),
            pltpu.VMEM((K,), jnp.int32),
            pltpu.VMEM((K,), jnp.int32),
            pltpu.VMEM((K,), jnp.int32),
            pltpu.VMEM((K, D), jnp.float32),
            pltpu.VMEM((K, D), jnp.float32),
        ],
    )
    def sc_kernel(hw_hbm, src_hbm, tgt_hbm, rel_hbm, relemb_hbm, out_hbm,
                  acc, sid, tid, rid, rows, rrows):
        c = lax.axis_index("c")
        s = lax.axis_index("s")

        @pl.loop(0, K)
        def _(e):
            for j in range(NSEG):
                rows[e, pl.ds(j * LN, LN)] = jnp.zeros((LN,), jnp.float32)
        for i in range(ZCH):
            pltpu.sync_copy(rows, acc.at[pl.ds((s * ZCH + i) * K, K)])
        plsc.subcore_barrier()

        ebase = s * EPT
        base_off = (c * N).astype(jnp.int32)

        @pl.loop(0, CH)
        def _(ch):
            g = ebase + ch * K
            pltpu.sync_copy(src_hbm.at[pl.ds(g, K)], sid)
            pltpu.sync_copy(tgt_hbm.at[pl.ds(g, K)], tid)
            pltpu.sync_copy(rel_hbm.at[pl.ds(g, K)], rid)
            for j in range(K // LN):
                sid[pl.ds(j * LN, LN)] = sid[pl.ds(j * LN, LN)] + base_off
            pltpu.sync_copy(hw_hbm.at[sid], rows)
            pltpu.sync_copy(relemb_hbm.at[rid], rrows)

            @pl.loop(0, K)
            def _(e):
                for j in range(NSEG):
                    rows[e, pl.ds(j * LN, LN)] = (
                        rows[e, pl.ds(j * LN, LN)] * rrows[e, pl.ds(j * LN, LN)])

            pltpu.sync_copy(rows, acc.at[tid], add=True)

        plsc.subcore_barrier()
        pltpu.sync_copy(acc.at[pl.ds(s * S_OUT, S_OUT)],
                        out_hbm.at[c, pl.ds(s * S_OUT, S_OUT)])
        if TAIL:
            @pl.when(s == 0)
            def _():
                pltpu.sync_copy(acc.at[pl.ds(NS * S_OUT, TAIL)],
                                out_hbm.at[c, pl.ds(NS * S_OUT, TAIL)])

    return sc_kernel


def kernel(h, edge_src, edge_tgt, edge_rel, nE, msg_W, rel_emb, update_W,
           update_b, ln_gamma, ln_beta):
    B, N, D = h.shape
    E = edge_src.shape[0]
    BLK = 500
    h2d = h.reshape(B * N, D)

    Wa = update_W[:, :D]
    Wb = update_W[:, D:]
    grid = (B * N // BLK,)
    hW, hU = pl.pallas_call(
        _mm2_body,
        grid=grid,
        in_specs=[
            pl.BlockSpec((BLK, D), lambda i: (i, 0)),
            pl.BlockSpec((D, D), lambda i: (0, 0)),
            pl.BlockSpec((D, D), lambda i: (0, 0)),
        ],
        out_specs=[
            pl.BlockSpec((BLK, D), lambda i: (i, 0)),
            pl.BlockSpec((BLK, D), lambda i: (i, 0)),
        ],
        out_shape=[
            jax.ShapeDtypeStruct((B * N, D), jnp.float32),
            jax.ShapeDtypeStruct((B * N, D), jnp.float32),
        ],
    )(h2d, msg_W, Wa)

    EPT_raw = E // NS
    CH = -(-EPT_raw // K)
    EPT = CH * K
    pad = EPT - EPT_raw
    ACC_ROWS = -(-(N + 1) // (NS * K)) * NS * K

    def _pad_edges(x, fill):
        x = x.astype(jnp.int32).reshape(NS, EPT_raw)
        return jnp.pad(x, ((0, 0), (0, pad)), constant_values=fill).reshape(-1)

    srcp = _pad_edges(edge_src, 0)
    tgtp = _pad_edges(edge_tgt, N)
    relp = _pad_edges(edge_rel, 0)

    sc = _make_sc_kernel(N, D, EPT, ACC_ROWS)
    h_agg = sc(hW, srcp, tgtp, relp, rel_emb)
    h_agg2d = h_agg.reshape(B * N, D)

    out = pl.pallas_call(
        _update_body,
        grid=grid,
        in_specs=[
            pl.BlockSpec((BLK, D), lambda i: (i, 0)),
            pl.BlockSpec((BLK, D), lambda i: (i, 0)),
            pl.BlockSpec((BLK, D), lambda i: (i, 0)),
            pl.BlockSpec((D, D), lambda i: (0, 0)),
            pl.BlockSpec((1, D), lambda i: (0, 0)),
            pl.BlockSpec((1, D), lambda i: (0, 0)),
            pl.BlockSpec((1, D), lambda i: (0, 0)),
        ],
        out_specs=pl.BlockSpec((BLK, D), lambda i: (i, 0)),
        out_shape=jax.ShapeDtypeStruct((B * N, D), jnp.float32),
    )(h2d, hU, h_agg2d, Wb, update_b.reshape(1, D), ln_gamma.reshape(1, D),
      ln_beta.reshape(1, D))
    return out.reshape(B, N, D)


# speedup vs baseline: 31.2429x; 31.2429x over previous
import functools

import jax
import jax.numpy as jnp
from jax import lax
from jax.experimental import pallas as pl
from jax.experimental.pallas import tpu as pltpu
from jax.experimental.pallas import tpu_sc as plsc

NC = 2
NS = 16
LN = 16
K = 128


def _mm2_body(h_ref, w1_ref, w2_ref, o1_ref, o2_ref):
    x = h_ref[...]
    dn = (((1,), (1,)), ((), ()))
    o1_ref[...] = lax.dot_general(x, w1_ref[...], dn,
                                  preferred_element_type=jnp.float32)
    o2_ref[...] = lax.dot_general(x, w2_ref[...], dn,
                                  preferred_element_type=jnp.float32)


def _update_body(h_ref, hu_ref, agg_ref, wb_ref, b_ref, g_ref, be_ref, o_ref):
    dn = (((1,), (1,)), ((), ()))
    u = hu_ref[...] + lax.dot_general(agg_ref[...], wb_ref[...], dn,
                                      preferred_element_type=jnp.float32)
    u = jnp.maximum(u + b_ref[...], 0.0)
    x = h_ref[...] + u
    mu = jnp.mean(x, axis=-1, keepdims=True)
    var = jnp.mean(jnp.square(x - mu), axis=-1, keepdims=True)
    o_ref[...] = (x - mu) * lax.rsqrt(var + 1e-5) * g_ref[...] + be_ref[...]


def _make_sc_kernel(N, D, EPT, ACC_ROWS):
    CH = EPT // K
    NSEG = D // LN
    ZCH = ACC_ROWS // (NS * K)
    S_OUT = (N // (8 * NS)) * 8
    TAIL = N - NS * S_OUT
    mesh = plsc.VectorSubcoreMesh(core_axis_name="c", subcore_axis_name="s")

    @functools.partial(
        pl.kernel,
        out_type=jax.ShapeDtypeStruct((NC, N, D), jnp.float32),
        mesh=mesh,
        scratch_types=[
            pltpu.VMEM_SHARED((ACC_ROWS, D), jnp.float32),
            pltpu.VMEM((K,), jnp.int32),
            pltpu.VMEM((K,), jnp.int32),
            pltpu.VMEM((K,), jnp.int32),
            pltpu.VMEM((K, D), jnp.float32),
            pltpu.VMEM((K, D), jnp.float32),
        ],
    )
    def sc_kernel(hw_hbm, src_hbm, tgt_hbm, rel_hbm, relemb_hbm, out_hbm,
                  acc, sid, tid, rid, rows, rrows):
        c = lax.axis_index("c")
        s = lax.axis_index("s")

        @pl.loop(0, K)
        def _(e):
            for j in range(NSEG):
                rows[e, pl.ds(j * LN, LN)] = jnp.zeros((LN,), jnp.float32)
        for i in range(ZCH):
            pltpu.sync_copy(rows, acc.at[pl.ds((s * ZCH + i) * K, K)])
        plsc.subcore_barrier()

        ebase = s * EPT
        base_off = (c * N).astype(jnp.int32)

        @pl.loop(0, CH)
        def _(ch):
            g = ebase + ch * K
            pltpu.sync_copy(src_hbm.at[pl.ds(g, K)], sid)
            pltpu.sync_copy(tgt_hbm.at[pl.ds(g, K)], tid)
            pltpu.sync_copy(rel_hbm.at[pl.ds(g, K)], rid)
            for j in range(K // LN):
                sid[pl.ds(j * LN, LN)] = sid[pl.ds(j * LN, LN)] + base_off
            pltpu.sync_copy(hw_hbm.at[sid], rows)
            pltpu.sync_copy(relemb_hbm.at[rid], rrows)

            @pl.loop(0, K)
            def _(e):
                for j in range(NSEG):
                    rows[e, pl.ds(j * LN, LN)] = (
                        rows[e, pl.ds(j * LN, LN)] * rrows[e, pl.ds(j * LN, LN)])

            pltpu.sync_copy(rows, acc.at[tid], add=True)

        plsc.subcore_barrier()
        pltpu.sync_copy(acc.at[pl.ds(s * S_OUT, S_OUT)],
                        out_hbm.at[c, pl.ds(s * S_OUT, S_OUT)])
        if TAIL:
            @pl.when(s == 0)
            def _():
                pltpu.sync_copy(acc.at[pl.ds(NS * S_OUT, TAIL)],
                                out_hbm.at[c, pl.ds(NS * S_OUT, TAIL)])

    return sc_kernel


def kernel(h, edge_src, edge_tgt, edge_rel, nE, msg_W, rel_emb, update_W,
           update_b, ln_gamma, ln_beta):
    B, N, D = h.shape
    E = edge_src.shape[0]
    BLK = 1000
    h2d = h.reshape(B * N, D)

    Wa = update_W[:, :D]
    Wb = update_W[:, D:]
    grid = (B * N // BLK,)
    hW, hU = pl.pallas_call(
        _mm2_body,
        grid=grid,
        in_specs=[
            pl.BlockSpec((BLK, D), lambda i: (i, 0)),
            pl.BlockSpec((D, D), lambda i: (0, 0)),
            pl.BlockSpec((D, D), lambda i: (0, 0)),
        ],
        out_specs=[
            pl.BlockSpec((BLK, D), lambda i: (i, 0)),
            pl.BlockSpec((BLK, D), lambda i: (i, 0)),
        ],
        out_shape=[
            jax.ShapeDtypeStruct((B * N, D), jnp.float32),
            jax.ShapeDtypeStruct((B * N, D), jnp.float32),
        ],
    )(h2d, msg_W, Wa)

    EPT_raw = E // NS
    CH = -(-EPT_raw // K)
    EPT = CH * K
    pad = EPT - EPT_raw
    ACC_ROWS = -(-(N + 1) // (NS * K)) * NS * K

    def _pad_edges(x, fill):
        x = x.astype(jnp.int32).reshape(NS, EPT_raw)
        return jnp.pad(x, ((0, 0), (0, pad)), constant_values=fill).reshape(-1)

    srcp = _pad_edges(edge_src, 0)
    tgtp = _pad_edges(edge_tgt, N)
    relp = _pad_edges(edge_rel, 0)

    sc = _make_sc_kernel(N, D, EPT, ACC_ROWS)
    h_agg = sc(hW, srcp, tgtp, relp, rel_emb)
    h_agg2d = h_agg.reshape(B * N, D)

    out = pl.pallas_call(
        _update_body,
        grid=grid,
        in_specs=[
            pl.BlockSpec((BLK, D), lambda i: (i, 0)),
            pl.BlockSpec((BLK, D), lambda i: (i, 0)),
            pl.BlockSpec((BLK, D), lambda i: (i, 0)),
            pl.BlockSpec((D, D), lambda i: (0, 0)),
            pl.BlockSpec((1, D), lambda i: (0, 0)),
            pl.BlockSpec((1, D), lambda i: (0, 0)),
            pl.BlockSpec((1, D), lambda i: (0, 0)),
        ],
        out_specs=pl.BlockSpec((BLK, D), lambda i: (i, 0)),
        out_shape=jax.ShapeDtypeStruct((B * N, D), jnp.float32),
    )(h2d, hU, h_agg2d, Wb, update_b.reshape(1, D), ln_gamma.reshape(1, D),
      ln_beta.reshape(1, D))
    return out.reshape(B, N, D)
